# baseline (speedup 1.0000x reference)
import functools

import jax
import jax.numpy as jnp
from jax import lax
from jax.experimental import pallas as pl
from jax.experimental.pallas import tpu as pltpu
from jax.experimental.pallas import tpu_sc as plsc

VIDEO_DIM = 1024
EMBED = 512
BATCH = 4096
HIST = 50
HISTP = 52
LANES = 16
SEG = 128
NSEG = EMBED // SEG
NC = 2
NS = 16
NW = NC * NS
BPW = BATCH // NW
NPAIR = BPW // 2
NJOB = NSEG * NPAIR
NBUF = 4
INV_HIST = 1.0 / HIST


@functools.partial(
    pl.kernel,
    out_type=jax.ShapeDtypeStruct((BATCH, EMBED), jnp.float32),
    mesh=plsc.VectorSubcoreMesh(core_axis_name="c", subcore_axis_name="s"),
    scratch_types=[
        pltpu.VMEM((BPW * HISTP,), jnp.int32),
        pltpu.VMEM((2 * HISTP, SEG), jnp.float32),
        pltpu.VMEM((2 * HISTP, SEG), jnp.float32),
        pltpu.VMEM((2 * HISTP, SEG), jnp.float32),
        pltpu.VMEM((2 * HISTP, SEG), jnp.float32),
        pltpu.VMEM((BPW, SEG), jnp.float32),
        pltpu.SemaphoreType.DMA,
        pltpu.SemaphoreType.DMA,
        pltpu.SemaphoreType.DMA,
        pltpu.SemaphoreType.DMA,
    ],
)
def _bag_kernel(table_hbm, idx_hbm, out_hbm, idx_v, g0, g1, g2, g3, slab,
                s0, s1, s2, s3):
    bufs = (g0, g1, g2, g3)
    sems = (s0, s1, s2, s3)
    wid = lax.axis_index("s") * NC + lax.axis_index("c")
    base = wid * BPW

    pltpu.sync_copy(idx_hbm.at[pl.ds(base * HISTP, BPW * HISTP)], idx_v)

    def start_job(t, par):
        seg = t // NPAIR
        pair = t % NPAIR
        pltpu.async_copy(
            table_hbm.at[idx_v.at[pl.ds(pair * (2 * HISTP), 2 * HISTP)],
                         pl.ds(seg * SEG, SEG)],
            bufs[par], sems[par])

    def wait_job(par):
        pltpu.make_async_copy(
            table_hbm.at[idx_v.at[pl.ds(0, 2 * HISTP)], pl.ds(0, SEG)],
            bufs[par], sems[par]).wait()

    for par in range(NBUF):
        start_job(par, par)

    @pl.loop(0, NJOB, step=NBUF)
    def _(T):
        for par in range(NBUF):
            t = T + par
            seg = t // NPAIR
            pair = t % NPAIR
            buf = bufs[par]
            wait_job(par)

            @pl.loop(0, SEG // LANES)
            def _(j):
                e = j * LANES
                for r in (0, 1):
                    ro = r * HISTP
                    accs = [buf[ro + i, pl.ds(e, LANES)] for i in range(4)]
                    for i in range(4, HIST):
                        k = i & 3
                        accs[k] = accs[k] + buf[ro + i, pl.ds(e, LANES)]
                    acc = (accs[0] + accs[1]) + (accs[2] + accs[3])
                    slab[2 * pair + r, pl.ds(e, LANES)] = acc * INV_HIST

            @pl.when(t + NBUF < NJOB)
            def _():
                start_job(t + NBUF, par)

            @pl.when(pair == NPAIR - 1)
            def _():
                pltpu.sync_copy(
                    slab,
                    out_hbm.at[pl.ds(base, BPW), pl.ds(seg * SEG, SEG)])


def _mm_body(v_ref, w_ref, b_ref, o_ref):
    o_ref[...] = (
        jnp.dot(v_ref[...], w_ref[...], preferred_element_type=jnp.float32,
                precision=lax.Precision.HIGHEST)
        + b_ref[...]
    )


def _video_embed(video, W, b):
    TM = 512
    return pl.pallas_call(
        _mm_body,
        grid=(BATCH // TM,),
        in_specs=[
            pl.BlockSpec((TM, VIDEO_DIM), lambda i: (i, 0)),
            pl.BlockSpec((VIDEO_DIM, EMBED), lambda i: (0, 0)),
            pl.BlockSpec((1, EMBED), lambda i: (0, 0)),
        ],
        out_specs=pl.BlockSpec((TM, EMBED), lambda i: (i, 0)),
        out_shape=jax.ShapeDtypeStruct((BATCH, EMBED), jnp.float32),
    )(video, W, b.reshape(1, EMBED))


def kernel(video, text, W, b, table):
    idx = text.astype(jnp.int32)
    idxp = jnp.pad(idx, ((0, 0), (0, HISTP - HIST))).reshape(-1)
    text_embed = _bag_kernel(table, idxp)
    video_embed = _video_embed(video, W, b)
    return jnp.concatenate([video_embed, text_embed], axis=-1)


# speedup vs baseline: 4.0623x; 4.0623x over previous
import functools

import jax
import jax.numpy as jnp
from jax import lax
from jax.experimental import pallas as pl
from jax.experimental.pallas import tpu as pltpu
from jax.experimental.pallas import tpu_sc as plsc

VIDEO_DIM = 1024
EMBED = 512
BATCH = 4096
HIST = 50
HISTP = 56
LANES = 16
SEG = 128
NSEG = EMBED // SEG
NC = 2
NS = 16
NW = NC * NS
BPW = BATCH // NW
NBUF = 3
OSLAB = 32
INV_HIST = 1.0 / HIST

_ROW_BUFS = [pltpu.VMEM((HIST, SEG), jnp.float32) for _ in range(NBUF * NSEG)]


@functools.partial(
    pl.kernel,
    out_type=jax.ShapeDtypeStruct((BATCH, EMBED), jnp.float32),
    mesh=plsc.VectorSubcoreMesh(core_axis_name="c", subcore_axis_name="s"),
    scratch_types=[
        pltpu.VMEM((BPW * HISTP,), jnp.int32),
        *_ROW_BUFS,
        pltpu.VMEM((OSLAB, EMBED), jnp.float32),
        pltpu.SemaphoreType.DMA,
        pltpu.SemaphoreType.DMA,
        pltpu.SemaphoreType.DMA,
    ],
)
def _bag_kernel(table_hbm, idx_hbm, out_hbm, idx_v,
                a0, a1, a2, a3, b0, b1, b2, b3, c0, c1, c2, c3,
                slab, s0, s1, s2):
    bufs = ((a0, a1, a2, a3), (b0, b1, b2, b3), (c0, c1, c2, c3))
    sems = (s0, s1, s2)
    wid = lax.axis_index("s") * NC + lax.axis_index("c")
    base = wid * BPW

    pltpu.sync_copy(idx_hbm.at[pl.ds(base * HISTP, BPW * HISTP)], idx_v)

    def start_gathers(b, par):
        for s in range(NSEG):
            pltpu.async_copy(
                table_hbm.at[idx_v.at[pl.ds(b * HISTP, HIST)],
                             pl.ds(s * SEG, SEG)],
                bufs[par][s], sems[par])

    def wait_gathers(par):
        for s in range(NSEG):
            pltpu.make_async_copy(
                table_hbm.at[idx_v.at[pl.ds(0, HIST)], pl.ds(0, SEG)],
                bufs[par][s], sems[par]).wait()

    def body(bb, par, prefetch):
        wait_gathers(par)
        srow = bb & (OSLAB - 1)

        for s in range(NSEG):
            buf = bufs[par][s]

            @pl.loop(0, SEG // LANES)
            def _(j):
                e = j * LANES
                accs = [buf[i, pl.ds(e, LANES)] for i in range(4)]
                for i in range(4, HIST):
                    k = i & 3
                    accs[k] = accs[k] + buf[i, pl.ds(e, LANES)]
                acc = (accs[0] + accs[1]) + (accs[2] + accs[3])
                slab[srow, pl.ds(s * SEG + e, LANES)] = acc * INV_HIST

        if prefetch:
            @pl.when(bb + NBUF < BPW)
            def _():
                start_gathers(bb + NBUF, par)

        @pl.when(srow == OSLAB - 1)
        def _():
            row0 = pl.multiple_of(base + bb - (OSLAB - 1), OSLAB)
            pltpu.sync_copy(slab, out_hbm.at[pl.ds(row0, OSLAB)])

    for par in range(NBUF):
        start_gathers(par, par)

    @pl.loop(0, BPW - 2, step=NBUF)
    def _(T):
        for par in range(NBUF):
            body(T + par, par, True)

    body(BPW - 2, 0, False)
    body(BPW - 1, 1, False)


def _mm_body(v_ref, w_ref, b_ref, o_ref):
    o_ref[...] = (
        jnp.dot(v_ref[...], w_ref[...], preferred_element_type=jnp.float32,
                precision=lax.Precision.HIGHEST)
        + b_ref[...]
    )


def _video_embed(video, W, b):
    TM = 512
    return pl.pallas_call(
        _mm_body,
        grid=(BATCH // TM,),
        in_specs=[
            pl.BlockSpec((TM, VIDEO_DIM), lambda i: (i, 0)),
            pl.BlockSpec((VIDEO_DIM, EMBED), lambda i: (0, 0)),
            pl.BlockSpec((1, EMBED), lambda i: (0, 0)),
        ],
        out_specs=pl.BlockSpec((TM, EMBED), lambda i: (i, 0)),
        out_shape=jax.ShapeDtypeStruct((BATCH, EMBED), jnp.float32),
    )(video, W, b.reshape(1, EMBED))


def kernel(video, text, W, b, table):
    idx = text.astype(jnp.int32)
    idxp = jnp.pad(idx, ((0, 0), (0, HISTP - HIST))).reshape(-1)
    text_embed = _bag_kernel(table, idxp)
    video_embed = _video_embed(video, W, b)
    return jnp.concatenate([video_embed, text_embed], axis=-1)


# speedup vs baseline: 4.0827x; 1.0050x over previous
import functools

import jax
import jax.numpy as jnp
from jax import lax
from jax.experimental import pallas as pl
from jax.experimental.pallas import tpu as pltpu
from jax.experimental.pallas import tpu_sc as plsc

VIDEO_DIM = 1024
EMBED = 512
BATCH = 4096
HIST = 50
HISTP = 56
LANES = 16
SEG = 128
NSEG = EMBED // SEG
NC = 2
NS = 16
NW = NC * NS
BPW = BATCH // NW
NBUF = 3
OSLAB = 32
INV_HIST = 1.0 / HIST

_ROW_BUFS = [pltpu.VMEM((HIST, SEG), jnp.float32) for _ in range(NBUF * NSEG)]


@functools.partial(
    pl.kernel,
    out_type=jax.ShapeDtypeStruct((BATCH, EMBED), jnp.float32),
    mesh=plsc.VectorSubcoreMesh(core_axis_name="c", subcore_axis_name="s"),
    scratch_types=[
        pltpu.VMEM((BPW * HISTP,), jnp.int32),
        *_ROW_BUFS,
        pltpu.VMEM((OSLAB, EMBED), jnp.float32),
        pltpu.SemaphoreType.DMA,
        pltpu.SemaphoreType.DMA,
        pltpu.SemaphoreType.DMA,
    ],
)
def _bag_kernel(table_hbm, idx_hbm, out_hbm, idx_v,
                a0, a1, a2, a3, b0, b1, b2, b3, c0, c1, c2, c3,
                slab, s0, s1, s2):
    bufs = ((a0, a1, a2, a3), (b0, b1, b2, b3), (c0, c1, c2, c3))
    sems = (s0, s1, s2)
    wid = lax.axis_index("s") * NC + lax.axis_index("c")
    base = wid * BPW

    pltpu.sync_copy(idx_hbm.at[pl.ds(base * HISTP, BPW * HISTP)], idx_v)

    def start_gather(b, par, s):
        pltpu.async_copy(
            table_hbm.at[idx_v.at[pl.ds(b * HISTP, HIST)],
                         pl.ds(s * SEG, SEG)],
            bufs[par][s], sems[par])

    def start_gathers(b, par):
        for s in range(NSEG):
            start_gather(b, par, s)

    def wait_gathers(par):
        for s in range(NSEG):
            pltpu.make_async_copy(
                table_hbm.at[idx_v.at[pl.ds(0, HIST)], pl.ds(0, SEG)],
                bufs[par][s], sems[par]).wait()

    def body(bb, par, prefetch):
        wait_gathers(par)
        srow = bb & (OSLAB - 1)

        for s in range(NSEG):
            buf = bufs[par][s]

            @pl.loop(0, SEG // LANES)
            def _(j):
                e = j * LANES
                accs = [buf[i, pl.ds(e, LANES)] for i in range(4)]
                for i in range(4, HIST):
                    k = i & 3
                    accs[k] = accs[k] + buf[i, pl.ds(e, LANES)]
                acc = (accs[0] + accs[1]) + (accs[2] + accs[3])
                slab[srow, pl.ds(s * SEG + e, LANES)] = acc * INV_HIST

            if prefetch:
                @pl.when(bb + NBUF < BPW)
                def _():
                    start_gather(bb + NBUF, par, s)

        @pl.when(srow == OSLAB - 1)
        def _():
            row0 = pl.multiple_of(base + bb - (OSLAB - 1), OSLAB)
            pltpu.sync_copy(slab, out_hbm.at[pl.ds(row0, OSLAB)])

    for par in range(NBUF):
        start_gathers(par, par)

    @pl.loop(0, BPW - 2, step=NBUF)
    def _(T):
        for par in range(NBUF):
            body(T + par, par, True)

    body(BPW - 2, 0, False)
    body(BPW - 1, 1, False)


def _mm_body(v_ref, w_ref, b_ref, o_ref):
    o_ref[...] = (
        jnp.dot(v_ref[...], w_ref[...], preferred_element_type=jnp.float32,
                precision=lax.Precision.HIGHEST)
        + b_ref[...]
    )


def _video_embed(video, W, b):
    TM = 512
    return pl.pallas_call(
        _mm_body,
        grid=(BATCH // TM,),
        in_specs=[
            pl.BlockSpec((TM, VIDEO_DIM), lambda i: (i, 0)),
            pl.BlockSpec((VIDEO_DIM, EMBED), lambda i: (0, 0)),
            pl.BlockSpec((1, EMBED), lambda i: (0, 0)),
        ],
        out_specs=pl.BlockSpec((TM, EMBED), lambda i: (i, 0)),
        out_shape=jax.ShapeDtypeStruct((BATCH, EMBED), jnp.float32),
    )(video, W, b.reshape(1, EMBED))


def kernel(video, text, W, b, table):
    idx = text.astype(jnp.int32)
    idxp = jnp.pad(idx, ((0, 0), (0, HISTP - HIST))).reshape(-1)
    text_embed = _bag_kernel(table, idxp)
    video_embed = _video_embed(video, W, b)
    return jnp.concatenate([video_embed, text_embed], axis=-1)
